# baseline (speedup 1.0000x reference)
import functools

import jax
import jax.numpy as jnp
from jax import lax
from jax.experimental import pallas as pl
from jax.experimental.pallas import tpu as pltpu
from jax.experimental.pallas import tpu_sc as plsc

_CHUNK = 128


def _gather_fn(B, D, nc, ns):
    nw = nc * ns
    b_per_w = B // nw
    n_chunks = b_per_w // _CHUNK
    mesh = plsc.VectorSubcoreMesh(core_axis_name="c", subcore_axis_name="s")

    @functools.partial(
        pl.kernel,
        mesh=mesh,
        out_type=jax.ShapeDtypeStruct((B, D), jnp.float32),
        scratch_types=[
            pltpu.VMEM((n_chunks, _CHUNK), jnp.int32),
            pltpu.VMEM((2, _CHUNK, D), jnp.float32),
            pltpu.SemaphoreType.DMA,
            pltpu.SemaphoreType.DMA,
        ],
    )
    def k(table_hbm, idx_hbm, out_hbm, idx_v, rows_v, sem_g, sem_w):
        wid = lax.axis_index("s") * nc + lax.axis_index("c")
        base = wid * b_per_w
        pltpu.sync_copy(idx_hbm.at[wid], idx_v)
        gathers = [
            pltpu.async_copy(table_hbm.at[idx_v.at[0]], rows_v.at[0], sem_g)
        ]
        writes = [None] * n_chunks
        for i in range(n_chunks):
            gathers[i].wait()
            if i > 0:
                writes[i - 1].wait()
            if i + 1 < n_chunks:
                gathers.append(
                    pltpu.async_copy(
                        table_hbm.at[idx_v.at[i + 1]],
                        rows_v.at[(i + 1) % 2],
                        sem_g,
                    )
                )
            writes[i] = pltpu.async_copy(
                rows_v.at[i % 2],
                out_hbm.at[pl.ds(base + i * _CHUNK, _CHUNK)],
                sem_w,
            )
        writes[n_chunks - 1].wait()

    return k


def kernel(seq, state_idx, embedding_table):
    B = state_idx.shape[0]
    D = embedding_table.shape[1]
    info = plsc.get_sparse_core_info()
    nc, ns = info.num_cores, info.num_subcores
    idx = state_idx.reshape(nc * ns, B // (nc * ns) // _CHUNK, _CHUNK)
    return _gather_fn(B, D, nc, ns)(embedding_table, idx)


# speedup vs baseline: 1.1871x; 1.1871x over previous
import functools

import jax
import jax.numpy as jnp
from jax import lax
from jax.experimental import pallas as pl
from jax.experimental.pallas import tpu as pltpu
from jax.experimental.pallas import tpu_sc as plsc

_CHUNK = 128


def _gather_fn(V, B, D, nc, ns):
    nw = nc * ns
    b_per_w = B // nw
    n_chunks = b_per_w // _CHUNK
    mesh = plsc.VectorSubcoreMesh(core_axis_name="c", subcore_axis_name="s")

    @functools.partial(
        pl.kernel,
        mesh=mesh,
        out_type=jax.ShapeDtypeStruct((B, D), jnp.float32),
        scratch_types=[
            pltpu.VMEM((n_chunks, _CHUNK), jnp.int32),
            pltpu.VMEM((2, _CHUNK, D), jnp.float32),
            pltpu.VMEM_SHARED((V, D), jnp.float32),
            pltpu.SemaphoreType.DMA,
            pltpu.SemaphoreType.DMA,
        ],
    )
    def k(table_hbm, idx_hbm, out_hbm, idx_v, rows_v, table_sp, sem_g, sem_w):
        cid = lax.axis_index("c")
        sid = lax.axis_index("s")
        wid = sid * nc + cid
        base = wid * b_per_w

        @pl.when(sid == 0)
        def _():
            pltpu.sync_copy(table_hbm, table_sp)

        pltpu.sync_copy(idx_hbm.at[wid], idx_v)
        plsc.subcore_barrier()

        gathers = [
            pltpu.async_copy(table_sp.at[idx_v.at[0]], rows_v.at[0], sem_g)
        ]
        writes = [None] * n_chunks
        for i in range(n_chunks):
            gathers[i].wait()
            if i > 0:
                writes[i - 1].wait()
            if i + 1 < n_chunks:
                gathers.append(
                    pltpu.async_copy(
                        table_sp.at[idx_v.at[i + 1]],
                        rows_v.at[(i + 1) % 2],
                        sem_g,
                    )
                )
            writes[i] = pltpu.async_copy(
                rows_v.at[i % 2],
                out_hbm.at[pl.ds(base + i * _CHUNK, _CHUNK)],
                sem_w,
            )
        writes[n_chunks - 1].wait()

    return k


def kernel(seq, state_idx, embedding_table):
    V, D = embedding_table.shape
    B = state_idx.shape[0]
    info = plsc.get_sparse_core_info()
    nc, ns = info.num_cores, info.num_subcores
    idx = state_idx.reshape(nc * ns, B // (nc * ns) // _CHUNK, _CHUNK)
    return _gather_fn(V, B, D, nc, ns)(embedding_table, idx)
